# baseline (speedup 1.0000x reference)
import functools

import jax
import jax.numpy as jnp
from jax import lax
from jax.experimental import pallas as pl
from jax.experimental.pallas import tpu as pltpu
from jax.experimental.pallas import tpu_sc as plsc

_B = 4096
_N = 128
_K = 64
_L = 16
_NW = 32
_RPW = _B // _NW
_RG = _RPW // _L
_CH = _N // _L
_NB = 256
_NG = 16
_U_ITERS = 18

_mesh = plsc.VectorSubcoreMesh(core_axis_name="c", subcore_axis_name="s")


def _splat_i32(val):
    return jnp.full((_L,), val, jnp.int32)


@functools.partial(
    pl.kernel,
    mesh=_mesh,
    compiler_params=pltpu.CompilerParams(needs_layout_passes=False),
    out_type=jax.ShapeDtypeStruct((_B, _N), jnp.float32),
    scratch_types=[
        pltpu.VMEM((_L * _N,), jnp.float32),
        pltpu.VMEM((2 * _N * _RPW,), jnp.float32),
        pltpu.VMEM((2 * _N, _N), jnp.float32),
        pltpu.VMEM((_NB * _L,), jnp.float32),
        pltpu.VMEM((_NG * _L,), jnp.float32),
        pltpu.VMEM((2 * _N, _L), jnp.float32),
        pltpu.VMEM((_RPW,), jnp.float32),
        pltpu.VMEM((_RPW,), jnp.float32),
        pltpu.VMEM((_RPW, _N), jnp.float32),
    ],
)
def _sc_spike(x_hbm, w_hbm, out_hbm, x_v, ut, vtab, hist, coarse, zbuf,
              u64lo, u64hi, out_v):
    wid = lax.axis_index("s") * 2 + lax.axis_index("c")
    base = wid * _RPW
    pltpu.sync_copy(w_hbm, vtab.at[pl.ds(0, _N)])

    ones = jnp.ones((_L,), jnp.float32)
    zeros = jnp.zeros((_L,), jnp.float32)
    lane = lax.iota(jnp.int32, _L)
    kf = jnp.float32(_K)

    for g in range(_RG):
        pltpu.sync_copy(x_hbm.at[pl.ds((base + g * _L) * _N, _L * _N)], x_v)

        def _build_u(i, _, g=g):
            xcol = plsc.load_gather(x_v, [lane * _N + _splat_i32(i)])
            ut[pl.ds(i * _RPW + g * _L, _L)] = jnp.maximum(3.0 + xcol, 0.0)
            ut[pl.ds((_N + i) * _RPW + g * _L, _L)] = jnp.maximum(3.0 - xcol, 0.0)
            return 0

        lax.fori_loop(0, _N, _build_u, 0)

    def _build_v(i, _):
        for c in range(_CH):
            wv = vtab[i, pl.ds(c * _L, _L)]
            vtab[i, pl.ds(c * _L, _L)] = jnp.maximum(3.0 + wv, 0.0)
            vtab[_N + i, pl.ds(c * _L, _L)] = jnp.maximum(3.0 - wv, 0.0)
        return 0

    lax.fori_loop(0, _N, _build_v, 0)

    vmm = []
    for c in range(_CH):
        def _mm(i, carry, c=c):
            mn, mx = carry
            vv = vtab[i, pl.ds(c * _L, _L)]
            return jnp.minimum(mn, vv), jnp.maximum(mx, vv)

        vmm.append(lax.fori_loop(
            0, 2 * _N, _mm,
            (jnp.full((_L,), 1e9, jnp.float32), jnp.full((_L,), -1e9, jnp.float32))))

    def _ugroup(g, _):
        def _umax(i, mx):
            return jnp.maximum(mx, ut[pl.ds(i * _RPW + g * _L, _L)])

        hi = lax.fori_loop(0, 2 * _N, _umax, zeros)
        lo = zeros

        def _ubis(_, carry):
            lo, hi = carry
            mid = 0.5 * (lo + hi)

            def _ucnt(i, acc):
                return acc + jnp.where(ut[pl.ds(i * _RPW + g * _L, _L)] <= mid, 1.0, 0.0)

            cnt = lax.fori_loop(0, 2 * _N, _ucnt, zeros)
            ge = cnt >= kf
            return jnp.where(ge, lo, mid), jnp.where(ge, mid, hi)

        lo, hi = lax.fori_loop(0, _U_ITERS, _ubis, (lo, hi))
        u64lo[pl.ds(g * _L, _L)] = lo
        u64hi[pl.ds(g * _L, _L)] = hi
        return 0

    lax.fori_loop(0, _RG, _ugroup, 0)

    def _row(r, _):
        rsp = _splat_i32(r)
        u_lo = plsc.load_gather(u64lo, [rsp])
        u_hi = plsc.load_gather(u64hi, [rsp])

        for c in range(_CH):
            vmn, vmx = vmm[c]
            lo0 = u_lo + vmn
            hi0 = u_hi + vmx
            width = jnp.maximum(hi0 - lo0, 1e-6)
            scale = jnp.float32(_NB) / width

            spikes = []
            for variant in range(2):
                off = variant * _N
                def _zero(j, _):
                    hist[pl.ds(j * _L, _L)] = zeros
                    return 0

                lax.fori_loop(0, _NB, _zero, 0)
                for g in range(_NG):
                    coarse[pl.ds(g * _L, _L)] = zeros

                def _scat(i, _, c=c, off=off, lo0=lo0, scale=scale, rsp=rsp):
                    ub = plsc.load_gather(ut, [_splat_i32(i * _RPW) + rsp])
                    row = lax.bitwise_and(i + off, 2 * _N - 1)
                    z = ub + vtab[row, pl.ds(c * _L, _L)]
                    zbuf[i] = z
                    q = (z - lo0) * scale
                    qi = jnp.clip(q.astype(jnp.int32), 0, _NB - 1)
                    plsc.addupdate_scatter(
                        hist, [lax.shift_left(qi, 4) + lane], ones)
                    plsc.addupdate_scatter(
                        coarse, [lax.bitwise_and(qi, 0xF0) + lane], ones)
                    return 0

                lax.fori_loop(0, 2 * _N, _scat, 0)

                cum = zeros
                gsel = _splat_i32(_NG - 1)
                for g in range(_NG):
                    cg = coarse[pl.ds(g * _L, _L)]
                    cum = cum + cg
                    gsel = jnp.minimum(gsel, jnp.where(cum >= kf, g, _NG - 1))
                cbefore = zeros
                for g in range(_NG - 1):
                    cbefore = cbefore + jnp.where(
                        g < gsel, coarse[pl.ds(g * _L, _L)], 0.0)

                cumf = cbefore
                bsel = _splat_i32(_NB - 1)
                gbase = gsel * _NG
                for k in range(_NG):
                    idx = gbase + k
                    hv = plsc.load_gather(hist, [lax.shift_left(idx, 4) + lane])
                    cumf = cumf + hv
                    bsel = jnp.minimum(bsel, jnp.where(cumf >= kf, idx, _NB - 1))

                t = lo0 + (bsel + 1).astype(jnp.float32) * (width * (1.0 / _NB))

                def _smin(i, acc, t=t):
                    return acc + jnp.minimum(zbuf[i], t)

                s_min = lax.fori_loop(0, 2 * _N, _smin, zeros)
                spikes.append((s_min - jnp.float32(2 * _N - _K) * t) * (1.0 / _K))

            out_v[r, pl.ds(c * _L, _L)] = spikes[0] - spikes[1]
        return 0

    lax.fori_loop(0, _RPW, _row, 0)
    pltpu.sync_copy(out_v, out_hbm.at[pl.ds(base, _RPW)])


@jax.jit
def kernel(inputp, W):
    return _sc_spike(inputp.reshape(-1), W)


# speedup vs baseline: 1.3459x; 1.3459x over previous
import functools

import jax
import jax.numpy as jnp
from jax import lax
from jax.experimental import pallas as pl
from jax.experimental.pallas import tpu as pltpu
from jax.experimental.pallas import tpu_sc as plsc

_B = 4096
_N = 128
_K = 64
_L = 16
_NW = 32
_RPW = _B // _NW
_RG = _RPW // _L
_CH = _N // _L
_NB = 256
_NG = 16
_U_ITERS = 18

_mesh = plsc.VectorSubcoreMesh(core_axis_name="c", subcore_axis_name="s")


def _splat_i32(val):
    return jnp.full((_L,), val, jnp.int32)


@functools.partial(
    pl.kernel,
    mesh=_mesh,
    compiler_params=pltpu.CompilerParams(needs_layout_passes=False),
    out_type=jax.ShapeDtypeStruct((_B, _N), jnp.float32),
    scratch_types=[
        pltpu.VMEM((_L * _N,), jnp.float32),
        pltpu.VMEM((2 * _N * _RPW,), jnp.float32),
        pltpu.VMEM((2 * _N, _N), jnp.float32),
        pltpu.VMEM((_NB * _L,), jnp.float32),
        pltpu.VMEM((_NG * _L,), jnp.float32),
        pltpu.VMEM((2 * _N, _L), jnp.float32),
        pltpu.VMEM((_RPW,), jnp.float32),
        pltpu.VMEM((_RPW,), jnp.float32),
        pltpu.VMEM((_RPW, _N), jnp.float32),
    ],
)
def _sc_spike(x_hbm, w_hbm, out_hbm, x_v, ut, vtab, hist, coarse, zbuf,
              u64lo, u64hi, out_v):
    wid = lax.axis_index("s") * 2 + lax.axis_index("c")
    base = wid * _RPW
    pltpu.sync_copy(w_hbm, vtab.at[pl.ds(0, _N)])

    ones = jnp.ones((_L,), jnp.float32)
    zeros = jnp.zeros((_L,), jnp.float32)
    lane = lax.iota(jnp.int32, _L)
    kf = jnp.float32(_K)

    for g in range(_RG):
        pltpu.sync_copy(x_hbm.at[pl.ds((base + g * _L) * _N, _L * _N)], x_v)

        def _build_u(i, _, g=g):
            xcol = plsc.load_gather(x_v, [lane * _N + _splat_i32(i)])
            ut[pl.ds(i * _RPW + g * _L, _L)] = jnp.maximum(3.0 + xcol, 0.0)
            ut[pl.ds((_N + i) * _RPW + g * _L, _L)] = jnp.maximum(3.0 - xcol, 0.0)
            return 0

        lax.fori_loop(0, _N, _build_u, 0, unroll=8)

    def _build_v(i, _):
        for c in range(_CH):
            wv = vtab[i, pl.ds(c * _L, _L)]
            vtab[i, pl.ds(c * _L, _L)] = jnp.maximum(3.0 + wv, 0.0)
            vtab[_N + i, pl.ds(c * _L, _L)] = jnp.maximum(3.0 - wv, 0.0)
        return 0

    lax.fori_loop(0, _N, _build_v, 0, unroll=8)

    vmm = []
    for c in range(_CH):
        def _mm(i, carry, c=c):
            mn, mx = carry
            vv = vtab[i, pl.ds(c * _L, _L)]
            return jnp.minimum(mn, vv), jnp.maximum(mx, vv)

        vmm.append(lax.fori_loop(
            0, 2 * _N, _mm,
            (jnp.full((_L,), 1e9, jnp.float32), jnp.full((_L,), -1e9, jnp.float32))))

    def _ugroup(g, _):
        def _umax(i, mx):
            return jnp.maximum(mx, ut[pl.ds(i * _RPW + g * _L, _L)])

        hi = lax.fori_loop(0, 2 * _N, _umax, zeros, unroll=16)
        lo = zeros

        def _ubis(_, carry):
            lo, hi = carry
            mid = 0.5 * (lo + hi)

            def _ucnt(i, acc):
                return acc + jnp.where(ut[pl.ds(i * _RPW + g * _L, _L)] <= mid, 1.0, 0.0)

            cnt = lax.fori_loop(0, 2 * _N, _ucnt, zeros, unroll=16)
            ge = cnt >= kf
            return jnp.where(ge, lo, mid), jnp.where(ge, mid, hi)

        lo, hi = lax.fori_loop(0, _U_ITERS, _ubis, (lo, hi))
        u64lo[pl.ds(g * _L, _L)] = lo
        u64hi[pl.ds(g * _L, _L)] = hi
        return 0

    lax.fori_loop(0, _RG, _ugroup, 0)

    def _row(r, _):
        rsp = _splat_i32(r)
        u_lo = plsc.load_gather(u64lo, [rsp])
        u_hi = plsc.load_gather(u64hi, [rsp])

        for c in range(_CH):
            vmn, vmx = vmm[c]
            lo0 = u_lo + vmn
            hi0 = u_hi + vmx
            width = jnp.maximum(hi0 - lo0, 1e-6)
            scale = jnp.float32(_NB) / width

            spikes = []
            for variant in range(2):
                off = variant * _N
                def _zero(j, _):
                    hist[pl.ds(j * _L, _L)] = zeros
                    return 0

                lax.fori_loop(0, _NB, _zero, 0, unroll=16)
                for g in range(_NG):
                    coarse[pl.ds(g * _L, _L)] = zeros

                def _scat(i, _, c=c, off=off, lo0=lo0, scale=scale, rsp=rsp):
                    ub = plsc.load_gather(ut, [_splat_i32(i * _RPW) + rsp])
                    row = lax.bitwise_and(i + off, 2 * _N - 1)
                    z = ub + vtab[row, pl.ds(c * _L, _L)]
                    zbuf[i] = z
                    q = (z - lo0) * scale
                    qi = jnp.clip(q.astype(jnp.int32), 0, _NB - 1)
                    plsc.addupdate_scatter(
                        hist, [lax.shift_left(qi, 4) + lane], ones)
                    plsc.addupdate_scatter(
                        coarse, [lax.bitwise_and(qi, 0xF0) + lane], ones)
                    return 0

                lax.fori_loop(0, 2 * _N, _scat, 0, unroll=8)

                cum = zeros
                gsel = _splat_i32(_NG - 1)
                for g in range(_NG):
                    cg = coarse[pl.ds(g * _L, _L)]
                    cum = cum + cg
                    gsel = jnp.minimum(gsel, jnp.where(cum >= kf, g, _NG - 1))
                cbefore = zeros
                for g in range(_NG - 1):
                    cbefore = cbefore + jnp.where(
                        g < gsel, coarse[pl.ds(g * _L, _L)], 0.0)

                cumf = cbefore
                bsel = _splat_i32(_NB - 1)
                gbase = gsel * _NG
                for k in range(_NG):
                    idx = gbase + k
                    hv = plsc.load_gather(hist, [lax.shift_left(idx, 4) + lane])
                    cumf = cumf + hv
                    bsel = jnp.minimum(bsel, jnp.where(cumf >= kf, idx, _NB - 1))

                t = lo0 + (bsel + 1).astype(jnp.float32) * (width * (1.0 / _NB))

                def _smin(i, acc, t=t):
                    return acc + jnp.minimum(zbuf[i], t)

                s_min = lax.fori_loop(0, 2 * _N, _smin, zeros, unroll=16)
                spikes.append((s_min - jnp.float32(2 * _N - _K) * t) * (1.0 / _K))

            out_v[r, pl.ds(c * _L, _L)] = spikes[0] - spikes[1]
        return 0

    lax.fori_loop(0, _RPW, _row, 0)
    pltpu.sync_copy(out_v, out_hbm.at[pl.ds(base, _RPW)])


@jax.jit
def kernel(inputp, W):
    return _sc_spike(inputp.reshape(-1), W)


# speedup vs baseline: 1.3914x; 1.0338x over previous
import functools

import jax
import jax.numpy as jnp
from jax import lax
from jax.experimental import pallas as pl
from jax.experimental.pallas import tpu as pltpu
from jax.experimental.pallas import tpu_sc as plsc

_B = 4096
_N = 128
_K = 64
_L = 16
_NW = 32
_RPW = _B // _NW
_RG = _RPW // _L
_CH = _N // _L
_NB = 256
_NG = 16
_U_ITERS = 18

_mesh = plsc.VectorSubcoreMesh(core_axis_name="c", subcore_axis_name="s")


def _splat_i32(val):
    return jnp.full((_L,), val, jnp.int32)


@functools.partial(
    pl.kernel,
    mesh=_mesh,
    compiler_params=pltpu.CompilerParams(needs_layout_passes=False),
    out_type=jax.ShapeDtypeStruct((_B, _N), jnp.float32),
    scratch_types=[
        pltpu.VMEM((_L * _N,), jnp.float32),
        pltpu.VMEM((2 * _N * _RPW,), jnp.float32),
        pltpu.VMEM((2 * _N, _N), jnp.float32),
        pltpu.VMEM((_NB * _L,), jnp.float32),
        pltpu.VMEM((2 * _N, _L), jnp.float32),
        pltpu.VMEM((_RPW,), jnp.float32),
        pltpu.VMEM((_RPW,), jnp.float32),
        pltpu.VMEM((_L, _N), jnp.float32),
    ],
)
def _sc_spike(x_hbm, w_hbm, out_hbm, x_v, ut, vtab, hist, zbuf,
              u64lo, u64hi, out_v):
    wid = lax.axis_index("s") * 2 + lax.axis_index("c")
    base = wid * _RPW
    pltpu.sync_copy(w_hbm, vtab.at[pl.ds(0, _N)])

    ones = jnp.ones((_L,), jnp.float32)
    zeros = jnp.zeros((_L,), jnp.float32)
    lane = lax.iota(jnp.int32, _L)
    kf = jnp.float32(_K)

    for g in range(_RG):
        pltpu.sync_copy(x_hbm.at[pl.ds((base + g * _L) * _N, _L * _N)], x_v)

        def _build_u(i, _, g=g):
            xcol = plsc.load_gather(x_v, [lane * _N + _splat_i32(i)])
            ut[pl.ds(i * _RPW + g * _L, _L)] = jnp.maximum(3.0 + xcol, 0.0)
            ut[pl.ds((_N + i) * _RPW + g * _L, _L)] = jnp.maximum(3.0 - xcol, 0.0)
            return 0

        lax.fori_loop(0, _N, _build_u, 0, unroll=8)

    def _build_v(i, _):
        for c in range(_CH):
            wv = vtab[i, pl.ds(c * _L, _L)]
            vtab[i, pl.ds(c * _L, _L)] = jnp.maximum(3.0 + wv, 0.0)
            vtab[_N + i, pl.ds(c * _L, _L)] = jnp.maximum(3.0 - wv, 0.0)
        return 0

    lax.fori_loop(0, _N, _build_v, 0, unroll=8)

    vmm = []
    for c in range(_CH):
        def _mm(i, carry, c=c):
            mn, mx = carry
            vv = vtab[i, pl.ds(c * _L, _L)]
            return jnp.minimum(mn, vv), jnp.maximum(mx, vv)

        vmm.append(lax.fori_loop(
            0, 2 * _N, _mm,
            (jnp.full((_L,), 1e9, jnp.float32), jnp.full((_L,), -1e9, jnp.float32))))

    _ACC = 4

    def _ugroup(g, _):
        def _umax(i, accs):
            return tuple(
                jnp.maximum(accs[j], ut[pl.ds((i + (2 * _N // _ACC) * j) * _RPW + g * _L, _L)])
                for j in range(_ACC))

        hi_accs = lax.fori_loop(0, 2 * _N // _ACC, _umax, (zeros,) * _ACC, unroll=2)
        hi = hi_accs[0]
        for j in range(1, _ACC):
            hi = jnp.maximum(hi, hi_accs[j])
        lo = zeros

        def _ubis(_, carry):
            lo, hi = carry
            mid = 0.5 * (lo + hi)

            def _ucnt(i, accs):
                return tuple(
                    accs[j] + jnp.where(
                        ut[pl.ds((i + (2 * _N // _ACC) * j) * _RPW + g * _L, _L)] <= mid, 1.0, 0.0)
                    for j in range(_ACC))

            accs = lax.fori_loop(0, 2 * _N // _ACC, _ucnt, (zeros,) * _ACC, unroll=2)
            cnt = accs[0]
            for j in range(1, _ACC):
                cnt = cnt + accs[j]
            ge = cnt >= kf
            return jnp.where(ge, lo, mid), jnp.where(ge, mid, hi)

        lo, hi = lax.fori_loop(0, _U_ITERS, _ubis, (lo, hi))
        u64lo[pl.ds(g * _L, _L)] = lo
        u64hi[pl.ds(g * _L, _L)] = hi
        return 0

    lax.fori_loop(0, _RG, _ugroup, 0)

    def _rowgrp(rg, _):
      def _row(rl, _, rg=rg):
        r = rg * _L + rl
        rsp = _splat_i32(r)
        u_lo = plsc.load_gather(u64lo, [rsp])
        u_hi = plsc.load_gather(u64hi, [rsp])

        for c in range(_CH):
            vmn, vmx = vmm[c]
            lo0 = u_lo + vmn
            hi0 = u_hi + vmx
            width = jnp.maximum(hi0 - lo0, 1e-6)
            scale = jnp.float32(_NB) / width

            def _variant(variant, s_prev, c=c, lo0=lo0, scale=scale,
                         width=width, rsp=rsp, rl=rl):
                off = variant * _N

                def _zero(j, _):
                    hist[pl.ds(j * _L, _L)] = zeros
                    return 0

                lax.fori_loop(0, _NB, _zero, 0, unroll=8)

                def _scat(i, _):
                    ub = plsc.load_gather(ut, [_splat_i32(i * _RPW) + rsp])
                    row = lax.bitwise_and(i + off, 2 * _N - 1)
                    z = ub + vtab[row, pl.ds(c * _L, _L)]
                    zbuf[i] = z
                    q = (z - lo0) * scale
                    qi = jnp.clip(q.astype(jnp.int32), 0, _NB - 1)
                    plsc.addupdate_scatter(
                        hist, [lax.shift_left(qi, 4) + lane], ones)
                    return 0

                lax.fori_loop(0, 2 * _N, _scat, 0, unroll=4)

                def _cscan(g, carry):
                    cum, gsel, cbefore = carry
                    gb = g * _NG * _L

                    def _gsum(k, accs):
                        a0, a1 = accs
                        return (a0 + hist[pl.ds(gb + 2 * k * _L, _L)],
                                a1 + hist[pl.ds(gb + (2 * k + 1) * _L, _L)])

                    s0, s1 = lax.fori_loop(0, _NG // 2, _gsum, (zeros, zeros),
                                           unroll=8)
                    newcum = cum + s0 + s1
                    first = (newcum >= kf) & (gsel >= _NG)
                    gsel = jnp.where(first, g, gsel)
                    cbefore = jnp.where(first, cum, cbefore)
                    return newcum, gsel, cbefore

                _, gsel, cbefore = lax.fori_loop(
                    0, _NG, _cscan, (zeros, _splat_i32(_NG), zeros))

                def _fscan(k, carry):
                    cumf, bsel = carry
                    idx = gsel * _NG + k
                    hv = plsc.load_gather(hist, [lax.shift_left(idx, 4) + lane])
                    newcum = cumf + hv
                    first = (newcum >= kf) & (bsel >= _NB)
                    bsel = jnp.where(first, idx, bsel)
                    return newcum, bsel

                _, bsel = lax.fori_loop(
                    0, _NG, _fscan, (cbefore, _splat_i32(_NB)))

                t = lo0 + (bsel + 1).astype(jnp.float32) * (width * (1.0 / _NB))

                def _smin(i, accs):
                    return tuple(
                        accs[j] + jnp.minimum(zbuf[i + (2 * _N // _ACC) * j], t)
                        for j in range(_ACC))

                s_accs = lax.fori_loop(
                    0, 2 * _N // _ACC, _smin, (zeros,) * _ACC, unroll=2)
                s_min = s_accs[0]
                for j in range(1, _ACC):
                    s_min = s_min + s_accs[j]
                spike = (s_min - jnp.float32(2 * _N - _K) * t) * (1.0 / _K)

                @pl.when(variant == 1)
                def _():
                    out_v[rl, pl.ds(c * _L, _L)] = s_prev - spike

                return spike

            lax.fori_loop(0, 2, _variant, zeros)
        return 0

      lax.fori_loop(0, _L, _row, 0)
      pltpu.sync_copy(out_v, out_hbm.at[pl.ds(base + rg * _L, _L)])
      return 0

    lax.fori_loop(0, _RG, _rowgrp, 0)


@jax.jit
def kernel(inputp, W):
    return _sc_spike(inputp.reshape(-1), W)


# speedup vs baseline: 10.9071x; 7.8390x over previous
import functools

import jax
import jax.numpy as jnp
from jax import lax
from jax.experimental import pallas as pl
from jax.experimental.pallas import tpu as pltpu
from jax.experimental.pallas import tpu_sc as plsc

_B = 4096
_N = 128
_K = 64
_SC_B = 512
_TC_B = _B - _SC_B


_ROWS = 32
_U_ITERS_TC = 14
_Z_ITERS = 7


def _tc_spike_sum(z, lo, hi):
    kf = jnp.float32(_K)
    for _ in range(_Z_ITERS):
        mid = 0.5 * (lo + hi)
        cnt = jnp.sum((z <= mid[:, None, :]).astype(jnp.float32), axis=1)
        ge = cnt >= kf
        hi = jnp.where(ge, mid, hi)
        lo = jnp.where(ge, lo, mid)
    t = hi[:, None, :]
    s_min = jnp.sum(jnp.minimum(z, t), axis=1)
    return (s_min - jnp.float32(2 * _N - _K) * hi) * (1.0 / _K)


def _tc_body(x_ref, w_ref, o_ref):
    x = x_ref[...]
    w = w_ref[...]
    a = jnp.maximum(3.0 + x, 0.0)
    b = jnp.maximum(3.0 - x, 0.0)
    p = jnp.maximum(3.0 + w, 0.0)
    m = jnp.maximum(3.0 - w, 0.0)
    u = jnp.concatenate([a, b], axis=1)

    u_hi = jnp.max(u, axis=1)
    u_lo = jnp.zeros_like(u_hi)
    kf = jnp.float32(_K)
    for _ in range(_U_ITERS_TC):
        mid = 0.5 * (u_lo + u_hi)
        cnt = jnp.sum((u <= mid[:, None]).astype(jnp.float32), axis=1)
        ge = cnt >= kf
        u_hi = jnp.where(ge, mid, u_hi)
        u_lo = jnp.where(ge, u_lo, mid)

    v_p = jnp.concatenate([p, m], axis=0)
    v_m = jnp.concatenate([m, p], axis=0)
    v_min = jnp.min(v_p, axis=0)[None, :]
    v_max = jnp.max(v_p, axis=0)[None, :]
    lo0 = u_lo[:, None] + v_min
    hi0 = u_hi[:, None] + v_max

    uu = u[:, :, None]
    s_plus = _tc_spike_sum(uu + v_p[None, :, :], lo0, hi0)
    s_minus = _tc_spike_sum(uu + v_m[None, :, :], lo0, hi0)
    o_ref[...] = s_plus - s_minus


def _tc_part(x, W):
    grid = _TC_B // _ROWS
    return pl.pallas_call(
        _tc_body,
        grid=(grid,),
        in_specs=[
            pl.BlockSpec((_ROWS, _N), lambda i: (i, 0)),
            pl.BlockSpec((_N, _N), lambda i: (0, 0)),
        ],
        out_specs=pl.BlockSpec((_ROWS, _N), lambda i: (i, 0)),
        out_shape=jax.ShapeDtypeStruct((_TC_B, _N), jnp.float32),
    )(x, W)


_L = 16
_NW = 32
_RPW = _SC_B // _NW
_RG = _RPW // _L
_CH = _N // _L
_NB = 256
_NG = 16
_U_ITERS_SC = 18
_ACC = 4

_mesh = plsc.VectorSubcoreMesh(core_axis_name="c", subcore_axis_name="s")


def _splat_i32(val):
    return jnp.full((_L,), val, jnp.int32)


@functools.partial(
    pl.kernel,
    mesh=_mesh,
    compiler_params=pltpu.CompilerParams(needs_layout_passes=False),
    out_type=jax.ShapeDtypeStruct((_SC_B, _N), jnp.float32),
    scratch_types=[
        pltpu.VMEM((_L * _N,), jnp.float32),
        pltpu.VMEM((2 * _N * _RPW,), jnp.float32),
        pltpu.VMEM((2 * _N, _N), jnp.float32),
        pltpu.VMEM((_NB * _L,), jnp.float32),
        pltpu.VMEM((2 * _N, _L), jnp.float32),
        pltpu.VMEM((_RPW,), jnp.float32),
        pltpu.VMEM((_RPW,), jnp.float32),
        pltpu.VMEM((_L, _N), jnp.float32),
    ],
)
def _sc_spike(x_hbm, w_hbm, out_hbm, x_v, ut, vtab, hist, zbuf,
              u64lo, u64hi, out_v):
    wid = lax.axis_index("s") * 2 + lax.axis_index("c")
    base = wid * _RPW
    pltpu.sync_copy(w_hbm, vtab.at[pl.ds(0, _N)])

    ones = jnp.ones((_L,), jnp.float32)
    zeros = jnp.zeros((_L,), jnp.float32)
    lane = lax.iota(jnp.int32, _L)
    kf = jnp.float32(_K)

    for g in range(_RG):
        pltpu.sync_copy(x_hbm.at[pl.ds((base + g * _L) * _N, _L * _N)], x_v)

        def _build_u(i, _, g=g):
            xcol = plsc.load_gather(x_v, [lane * _N + _splat_i32(i)])
            ut[pl.ds(i * _RPW + g * _L, _L)] = jnp.maximum(3.0 + xcol, 0.0)
            ut[pl.ds((_N + i) * _RPW + g * _L, _L)] = jnp.maximum(3.0 - xcol, 0.0)
            return 0

        lax.fori_loop(0, _N, _build_u, 0, unroll=8)

    def _build_v(i, _):
        for c in range(_CH):
            wv = vtab[i, pl.ds(c * _L, _L)]
            vtab[i, pl.ds(c * _L, _L)] = jnp.maximum(3.0 + wv, 0.0)
            vtab[_N + i, pl.ds(c * _L, _L)] = jnp.maximum(3.0 - wv, 0.0)
        return 0

    lax.fori_loop(0, _N, _build_v, 0, unroll=8)

    vmm = []
    for c in range(_CH):
        def _mm(i, carry, c=c):
            mn, mx = carry
            vv = vtab[i, pl.ds(c * _L, _L)]
            return jnp.minimum(mn, vv), jnp.maximum(mx, vv)

        vmm.append(lax.fori_loop(
            0, 2 * _N, _mm,
            (jnp.full((_L,), 1e9, jnp.float32), jnp.full((_L,), -1e9, jnp.float32))))

    def _ugroup(g, _):
        def _umax(i, accs):
            return tuple(
                jnp.maximum(accs[j], ut[pl.ds((i + (2 * _N // _ACC) * j) * _RPW + g * _L, _L)])
                for j in range(_ACC))

        hi_accs = lax.fori_loop(0, 2 * _N // _ACC, _umax, (zeros,) * _ACC, unroll=2)
        hi = hi_accs[0]
        for j in range(1, _ACC):
            hi = jnp.maximum(hi, hi_accs[j])
        lo = zeros

        def _ubis(_, carry):
            lo, hi = carry
            mid = 0.5 * (lo + hi)

            def _ucnt(i, accs):
                return tuple(
                    accs[j] + jnp.where(
                        ut[pl.ds((i + (2 * _N // _ACC) * j) * _RPW + g * _L, _L)] <= mid, 1.0, 0.0)
                    for j in range(_ACC))

            accs = lax.fori_loop(0, 2 * _N // _ACC, _ucnt, (zeros,) * _ACC, unroll=2)
            cnt = accs[0]
            for j in range(1, _ACC):
                cnt = cnt + accs[j]
            ge = cnt >= kf
            return jnp.where(ge, lo, mid), jnp.where(ge, mid, hi)

        lo, hi = lax.fori_loop(0, _U_ITERS_SC, _ubis, (lo, hi))
        u64lo[pl.ds(g * _L, _L)] = lo
        u64hi[pl.ds(g * _L, _L)] = hi
        return 0

    lax.fori_loop(0, _RG, _ugroup, 0)

    def _rowgrp(rg, _):
      def _row(rl, _, rg=rg):
        r = rg * _L + rl
        rsp = _splat_i32(r)
        u_lo = plsc.load_gather(u64lo, [rsp])
        u_hi = plsc.load_gather(u64hi, [rsp])

        for c in range(_CH):
            vmn, vmx = vmm[c]
            lo0 = u_lo + vmn
            hi0 = u_hi + vmx
            width = jnp.maximum(hi0 - lo0, 1e-6)
            scale = jnp.float32(_NB) / width

            def _variant(variant, s_prev, c=c, lo0=lo0, scale=scale,
                         width=width, rsp=rsp, rl=rl):
                off = variant * _N

                def _zero(j, _):
                    hist[pl.ds(j * _L, _L)] = zeros
                    return 0

                lax.fori_loop(0, _NB, _zero, 0, unroll=8)

                def _scat(i, _):
                    ub = plsc.load_gather(ut, [_splat_i32(i * _RPW) + rsp])
                    row = lax.bitwise_and(i + off, 2 * _N - 1)
                    z = ub + vtab[row, pl.ds(c * _L, _L)]
                    zbuf[i] = z
                    q = (z - lo0) * scale
                    qi = jnp.clip(q.astype(jnp.int32), 0, _NB - 1)
                    plsc.addupdate_scatter(
                        hist, [lax.shift_left(qi, 4) + lane], ones)
                    return 0

                lax.fori_loop(0, 2 * _N, _scat, 0, unroll=4)

                def _cscan(g, carry):
                    cum, gsel, cbefore = carry
                    gb = g * _NG * _L

                    def _gsum(k, accs):
                        a0, a1 = accs
                        return (a0 + hist[pl.ds(gb + 2 * k * _L, _L)],
                                a1 + hist[pl.ds(gb + (2 * k + 1) * _L, _L)])

                    s0, s1 = lax.fori_loop(0, _NG // 2, _gsum, (zeros, zeros),
                                           unroll=8)
                    newcum = cum + s0 + s1
                    first = (newcum >= kf) & (gsel >= _NG)
                    gsel = jnp.where(first, g, gsel)
                    cbefore = jnp.where(first, cum, cbefore)
                    return newcum, gsel, cbefore

                _, gsel, cbefore = lax.fori_loop(
                    0, _NG, _cscan, (zeros, _splat_i32(_NG), zeros))

                def _fscan(k, carry):
                    cumf, bsel = carry
                    idx = gsel * _NG + k
                    hv = plsc.load_gather(hist, [lax.shift_left(idx, 4) + lane])
                    newcum = cumf + hv
                    first = (newcum >= kf) & (bsel >= _NB)
                    bsel = jnp.where(first, idx, bsel)
                    return newcum, bsel

                _, bsel = lax.fori_loop(
                    0, _NG, _fscan, (cbefore, _splat_i32(_NB)))

                t = lo0 + (bsel + 1).astype(jnp.float32) * (width * (1.0 / _NB))

                def _smin(i, accs):
                    return tuple(
                        accs[j] + jnp.minimum(zbuf[i + (2 * _N // _ACC) * j], t)
                        for j in range(_ACC))

                s_accs = lax.fori_loop(
                    0, 2 * _N // _ACC, _smin, (zeros,) * _ACC, unroll=2)
                s_min = s_accs[0]
                for j in range(1, _ACC):
                    s_min = s_min + s_accs[j]
                spike = (s_min - jnp.float32(2 * _N - _K) * t) * (1.0 / _K)

                @pl.when(variant == 1)
                def _():
                    out_v[rl, pl.ds(c * _L, _L)] = s_prev - spike

                return spike

            lax.fori_loop(0, 2, _variant, zeros)
        return 0

      lax.fori_loop(0, _L, _row, 0)
      pltpu.sync_copy(out_v, out_hbm.at[pl.ds(base + rg * _L, _L)])
      return 0

    lax.fori_loop(0, _RG, _rowgrp, 0)


@jax.jit
def kernel(inputp, W):
    out_sc = _sc_spike(inputp[_TC_B:].reshape(-1), W)
    out_tc = _tc_part(inputp[:_TC_B], W)
    return jnp.concatenate([out_tc, out_sc], axis=0)


# speedup vs baseline: 11.5902x; 1.0626x over previous
import functools

import jax
import jax.numpy as jnp
from jax import lax
from jax.experimental import pallas as pl
from jax.experimental.pallas import tpu as pltpu
from jax.experimental.pallas import tpu_sc as plsc

_B = 4096
_N = 128
_K = 64
_SC_B = 512
_TC_B = _B - _SC_B


_ROWS = 32
_U_ITERS_TC = 14
_Z_ITERS = 7


def _tc_spike_sum(z, lo, hi):
    kf = jnp.float32(_K)
    for _ in range(_Z_ITERS):
        mid = 0.5 * (lo + hi)
        cnt = jnp.sum((z <= mid[:, None, :]).astype(jnp.float32), axis=1)
        ge = cnt >= kf
        hi = jnp.where(ge, mid, hi)
        lo = jnp.where(ge, lo, mid)
    t = hi[:, None, :]
    s_min = jnp.sum(jnp.minimum(z, t), axis=1)
    return (s_min - jnp.float32(2 * _N - _K) * hi) * (1.0 / _K)


def _tc_body(x_ref, w_ref, o_ref):
    x = x_ref[...]
    w = w_ref[...]
    a = jnp.maximum(3.0 + x, 0.0)
    b = jnp.maximum(3.0 - x, 0.0)
    p = jnp.maximum(3.0 + w, 0.0)
    m = jnp.maximum(3.0 - w, 0.0)
    u = jnp.concatenate([a, b], axis=1)

    u_hi = jnp.max(u, axis=1)
    u_lo = jnp.zeros_like(u_hi)
    kf = jnp.float32(_K)
    for _ in range(_U_ITERS_TC):
        mid = 0.5 * (u_lo + u_hi)
        cnt = jnp.sum((u <= mid[:, None]).astype(jnp.float32), axis=1)
        ge = cnt >= kf
        u_hi = jnp.where(ge, mid, u_hi)
        u_lo = jnp.where(ge, u_lo, mid)

    v_p = jnp.concatenate([p, m], axis=0)
    v_m = jnp.concatenate([m, p], axis=0)
    v_min = jnp.min(v_p, axis=0)[None, :]
    v_max = jnp.max(v_p, axis=0)[None, :]
    lo0 = u_lo[:, None] + v_min
    hi0 = u_hi[:, None] + v_max

    uu = u[:, :, None]
    s_plus = _tc_spike_sum(uu + v_p[None, :, :], lo0, hi0)
    s_minus = _tc_spike_sum(uu + v_m[None, :, :], lo0, hi0)
    o_ref[...] = s_plus - s_minus


def _tc_part(x, W):
    grid = _TC_B // _ROWS
    return pl.pallas_call(
        _tc_body,
        grid=(grid,),
        in_specs=[
            pl.BlockSpec((_ROWS, _N), lambda i: (i, 0)),
            pl.BlockSpec((_N, _N), lambda i: (0, 0)),
        ],
        out_specs=pl.BlockSpec((_ROWS, _N), lambda i: (i, 0)),
        out_shape=jax.ShapeDtypeStruct((_TC_B, _N), jnp.float32),
    )(x, W)


_L = 16
_NW = 32
_RPW = _SC_B // _NW
_RG = _RPW // _L
_CH = _N // _L
_NB = 128
_NG = 16
_GS = _NB // _NG
_U_ITERS_SC = 12
_ACC = 4

_mesh = plsc.VectorSubcoreMesh(core_axis_name="c", subcore_axis_name="s")


def _splat_i32(val):
    return jnp.full((_L,), val, jnp.int32)


@functools.partial(
    pl.kernel,
    mesh=_mesh,
    compiler_params=pltpu.CompilerParams(needs_layout_passes=False),
    out_type=jax.ShapeDtypeStruct((_SC_B, _N), jnp.float32),
    scratch_types=[
        pltpu.VMEM((_L * _N,), jnp.float32),
        pltpu.VMEM((2 * _N * _RPW,), jnp.float32),
        pltpu.VMEM((2 * _N, _N), jnp.float32),
        pltpu.VMEM((_NB * _L,), jnp.float32),
        pltpu.VMEM((2 * _N, _L), jnp.float32),
        pltpu.VMEM((_RPW,), jnp.float32),
        pltpu.VMEM((_RPW,), jnp.float32),
        pltpu.VMEM((_L, _N), jnp.float32),
    ],
)
def _sc_spike(x_hbm, w_hbm, out_hbm, x_v, ut, vtab, hist, zbuf,
              u64lo, u64hi, out_v):
    wid = lax.axis_index("s") * 2 + lax.axis_index("c")
    base = wid * _RPW
    pltpu.sync_copy(w_hbm, vtab.at[pl.ds(0, _N)])

    ones = jnp.ones((_L,), jnp.float32)
    zeros = jnp.zeros((_L,), jnp.float32)
    lane = lax.iota(jnp.int32, _L)
    kf = jnp.float32(_K)

    for g in range(_RG):
        pltpu.sync_copy(x_hbm.at[pl.ds((base + g * _L) * _N, _L * _N)], x_v)

        def _build_u(i, _, g=g):
            xcol = plsc.load_gather(x_v, [lane * _N + _splat_i32(i)])
            ut[pl.ds(i * _RPW + g * _L, _L)] = jnp.maximum(3.0 + xcol, 0.0)
            ut[pl.ds((_N + i) * _RPW + g * _L, _L)] = jnp.maximum(3.0 - xcol, 0.0)
            return 0

        lax.fori_loop(0, _N, _build_u, 0, unroll=8)

    def _build_v(i, _):
        for c in range(_CH):
            wv = vtab[i, pl.ds(c * _L, _L)]
            vtab[i, pl.ds(c * _L, _L)] = jnp.maximum(3.0 + wv, 0.0)
            vtab[_N + i, pl.ds(c * _L, _L)] = jnp.maximum(3.0 - wv, 0.0)
        return 0

    lax.fori_loop(0, _N, _build_v, 0, unroll=8)

    vmm = []
    for c in range(_CH):
        def _mm(i, carry, c=c):
            mn, mx = carry
            vv = vtab[i, pl.ds(c * _L, _L)]
            return jnp.minimum(mn, vv), jnp.maximum(mx, vv)

        vmm.append(lax.fori_loop(
            0, 2 * _N, _mm,
            (jnp.full((_L,), 1e9, jnp.float32), jnp.full((_L,), -1e9, jnp.float32))))

    def _ugroup(g, _):
        def _umax(i, accs):
            return tuple(
                jnp.maximum(accs[j], ut[pl.ds((i + (2 * _N // _ACC) * j) * _RPW + g * _L, _L)])
                for j in range(_ACC))

        hi_accs = lax.fori_loop(0, 2 * _N // _ACC, _umax, (zeros,) * _ACC, unroll=2)
        hi = hi_accs[0]
        for j in range(1, _ACC):
            hi = jnp.maximum(hi, hi_accs[j])
        lo = zeros

        def _ubis(_, carry):
            lo, hi = carry
            mid = 0.5 * (lo + hi)

            def _ucnt(i, accs):
                return tuple(
                    accs[j] + jnp.where(
                        ut[pl.ds((i + (2 * _N // _ACC) * j) * _RPW + g * _L, _L)] <= mid, 1.0, 0.0)
                    for j in range(_ACC))

            accs = lax.fori_loop(0, 2 * _N // _ACC, _ucnt, (zeros,) * _ACC, unroll=2)
            cnt = accs[0]
            for j in range(1, _ACC):
                cnt = cnt + accs[j]
            ge = cnt >= kf
            return jnp.where(ge, lo, mid), jnp.where(ge, mid, hi)

        lo, hi = lax.fori_loop(0, _U_ITERS_SC, _ubis, (lo, hi))
        u64lo[pl.ds(g * _L, _L)] = lo
        u64hi[pl.ds(g * _L, _L)] = hi
        return 0

    lax.fori_loop(0, _RG, _ugroup, 0)

    def _rowgrp(rg, _):
      def _row(rl, _, rg=rg):
        r = rg * _L + rl
        rsp = _splat_i32(r)
        u_lo = plsc.load_gather(u64lo, [rsp])
        u_hi = plsc.load_gather(u64hi, [rsp])

        for c in range(_CH):
            vmn, vmx = vmm[c]
            lo0 = u_lo + vmn
            hi0 = u_hi + vmx
            width = jnp.maximum(hi0 - lo0, 1e-6)
            scale = jnp.float32(_NB) / width

            def _variant(variant, s_prev, c=c, lo0=lo0, scale=scale,
                         width=width, rsp=rsp, rl=rl):
                off = variant * _N

                def _zero(j, _):
                    hist[pl.ds(j * _L, _L)] = zeros
                    return 0

                lax.fori_loop(0, _NB, _zero, 0, unroll=8)

                def _scat(i, _):
                    ub = plsc.load_gather(ut, [_splat_i32(i * _RPW) + rsp])
                    row = lax.bitwise_and(i + off, 2 * _N - 1)
                    z = ub + vtab[row, pl.ds(c * _L, _L)]
                    zbuf[i] = z
                    q = (z - lo0) * scale
                    qi = jnp.clip(q.astype(jnp.int32), 0, _NB - 1)
                    plsc.addupdate_scatter(
                        hist, [lax.shift_left(qi, 4) + lane], ones)
                    return 0

                lax.fori_loop(0, 2 * _N, _scat, 0, unroll=8)

                def _cscan(g, carry):
                    cum, gsel, cbefore = carry
                    gb = g * _GS * _L

                    def _gsum(k, accs):
                        a0, a1 = accs
                        return (a0 + hist[pl.ds(gb + 2 * k * _L, _L)],
                                a1 + hist[pl.ds(gb + (2 * k + 1) * _L, _L)])

                    s0, s1 = lax.fori_loop(0, _GS // 2, _gsum, (zeros, zeros),
                                           unroll=4)
                    newcum = cum + s0 + s1
                    first = (newcum >= kf) & (gsel >= _NG)
                    gsel = jnp.where(first, g, gsel)
                    cbefore = jnp.where(first, cum, cbefore)
                    return newcum, gsel, cbefore

                _, gsel, cbefore = lax.fori_loop(
                    0, _NG, _cscan, (zeros, _splat_i32(_NG), zeros))

                def _fscan(k, carry):
                    cumf, bsel = carry
                    idx = gsel * _GS + k
                    hv = plsc.load_gather(hist, [lax.shift_left(idx, 4) + lane])
                    newcum = cumf + hv
                    first = (newcum >= kf) & (bsel >= _NB)
                    bsel = jnp.where(first, idx, bsel)
                    return newcum, bsel

                _, bsel = lax.fori_loop(
                    0, _GS, _fscan, (cbefore, _splat_i32(_NB)))

                t = lo0 + (bsel + 1).astype(jnp.float32) * (width * (1.0 / _NB))

                def _smin(i, accs):
                    return tuple(
                        accs[j] + jnp.minimum(zbuf[i + (2 * _N // _ACC) * j], t)
                        for j in range(_ACC))

                s_accs = lax.fori_loop(
                    0, 2 * _N // _ACC, _smin, (zeros,) * _ACC, unroll=2)
                s_min = s_accs[0]
                for j in range(1, _ACC):
                    s_min = s_min + s_accs[j]
                spike = (s_min - jnp.float32(2 * _N - _K) * t) * (1.0 / _K)

                @pl.when(variant == 1)
                def _():
                    out_v[rl, pl.ds(c * _L, _L)] = s_prev - spike

                return spike

            lax.fori_loop(0, 2, _variant, zeros)
        return 0

      lax.fori_loop(0, _L, _row, 0)
      pltpu.sync_copy(out_v, out_hbm.at[pl.ds(base + rg * _L, _L)])
      return 0

    lax.fori_loop(0, _RG, _rowgrp, 0)


@jax.jit
def kernel(inputp, W):
    out_sc = _sc_spike(inputp[_TC_B:].reshape(-1), W)
    out_tc = _tc_part(inputp[:_TC_B], W)
    return jnp.concatenate([out_tc, out_sc], axis=0)


# speedup vs baseline: 11.6624x; 1.0062x over previous
import functools

import jax
import jax.numpy as jnp
from jax import lax
from jax.experimental import pallas as pl
from jax.experimental.pallas import tpu as pltpu
from jax.experimental.pallas import tpu_sc as plsc

_B = 4096
_N = 128
_K = 64
_SC_B = 512
_TC_B = _B - _SC_B


_ROWS = 32
_U_ITERS_TC = 14
_Z_ITERS = 7


def _tc_spike_sum(z, lo, hi):
    kf = jnp.float32(_K)
    for _ in range(_Z_ITERS):
        mid = 0.5 * (lo + hi)
        cnt = jnp.sum((z <= mid[:, None, :]).astype(jnp.float32), axis=1)
        ge = cnt >= kf
        hi = jnp.where(ge, mid, hi)
        lo = jnp.where(ge, lo, mid)
    t = hi[:, None, :]
    s_min = jnp.sum(jnp.minimum(z, t), axis=1)
    return (s_min - jnp.float32(2 * _N - _K) * hi) * (1.0 / _K)


def _tc_body(x_ref, w_ref, o_ref):
    x = x_ref[...]
    w = w_ref[...]
    a = jnp.maximum(3.0 + x, 0.0)
    b = jnp.maximum(3.0 - x, 0.0)
    p = jnp.maximum(3.0 + w, 0.0)
    m = jnp.maximum(3.0 - w, 0.0)
    u = jnp.concatenate([a, b], axis=1)

    u_hi = jnp.max(u, axis=1)
    u_lo = jnp.zeros_like(u_hi)
    kf = jnp.float32(_K)
    for _ in range(_U_ITERS_TC):
        mid = 0.5 * (u_lo + u_hi)
        cnt = jnp.sum((u <= mid[:, None]).astype(jnp.float32), axis=1)
        ge = cnt >= kf
        u_hi = jnp.where(ge, mid, u_hi)
        u_lo = jnp.where(ge, u_lo, mid)

    v_p = jnp.concatenate([p, m], axis=0)
    v_m = jnp.concatenate([m, p], axis=0)
    v_min = jnp.min(v_p, axis=0)[None, :]
    v_max = jnp.max(v_p, axis=0)[None, :]
    lo0 = u_lo[:, None] + v_min
    hi0 = u_hi[:, None] + v_max

    uu = u[:, :, None]
    s_plus = _tc_spike_sum(uu + v_p[None, :, :], lo0, hi0)
    s_minus = _tc_spike_sum(uu + v_m[None, :, :], lo0, hi0)
    o_ref[...] = s_plus - s_minus


def _tc_part(x, W):
    grid = _TC_B // _ROWS
    return pl.pallas_call(
        _tc_body,
        grid=(grid,),
        in_specs=[
            pl.BlockSpec((_ROWS, _N), lambda i: (i, 0)),
            pl.BlockSpec((_N, _N), lambda i: (0, 0)),
        ],
        out_specs=pl.BlockSpec((_ROWS, _N), lambda i: (i, 0)),
        out_shape=jax.ShapeDtypeStruct((_TC_B, _N), jnp.float32),
    )(x, W)


_L = 16
_NW = 32
_RPW = _SC_B // _NW
_RG = _RPW // _L
_CH = _N // _L
_NB = 128
_NG = 16
_GS = _NB // _NG
_U_ITERS_SC = 12
_ACC = 4

_mesh = plsc.VectorSubcoreMesh(core_axis_name="c", subcore_axis_name="s")


def _splat_i32(val):
    return jnp.full((_L,), val, jnp.int32)


@functools.partial(
    pl.kernel,
    mesh=_mesh,
    compiler_params=pltpu.CompilerParams(needs_layout_passes=False),
    out_type=jax.ShapeDtypeStruct((_SC_B, _N), jnp.float32),
    scratch_types=[
        pltpu.VMEM((_L * _N,), jnp.float32),
        pltpu.VMEM((2 * _N * _RPW,), jnp.float32),
        pltpu.VMEM((2 * _N, _N), jnp.float32),
        pltpu.VMEM((_NB * _L,), jnp.float32),
        pltpu.VMEM((2 * _N, _L), jnp.float32),
        pltpu.VMEM((2 * _N, _L), jnp.float32),
        pltpu.VMEM((_RPW,), jnp.float32),
        pltpu.VMEM((_RPW,), jnp.float32),
        pltpu.VMEM((_L, _N), jnp.float32),
    ],
)
def _sc_spike(x_hbm, w_hbm, out_hbm, x_v, ut, vtab, hist, zbuf, ubc,
              u64lo, u64hi, out_v):
    wid = lax.axis_index("s") * 2 + lax.axis_index("c")
    base = wid * _RPW
    pltpu.sync_copy(w_hbm, vtab.at[pl.ds(0, _N)])

    ones = jnp.ones((_L,), jnp.float32)
    zeros = jnp.zeros((_L,), jnp.float32)
    lane = lax.iota(jnp.int32, _L)
    kf = jnp.float32(_K)

    for g in range(_RG):
        pltpu.sync_copy(x_hbm.at[pl.ds((base + g * _L) * _N, _L * _N)], x_v)

        def _build_u(i, _, g=g):
            xcol = plsc.load_gather(x_v, [lane * _N + _splat_i32(i)])
            ut[pl.ds(i * _RPW + g * _L, _L)] = jnp.maximum(3.0 + xcol, 0.0)
            ut[pl.ds((_N + i) * _RPW + g * _L, _L)] = jnp.maximum(3.0 - xcol, 0.0)
            return 0

        lax.fori_loop(0, _N, _build_u, 0, unroll=8)

    def _build_v(i, _):
        for c in range(_CH):
            wv = vtab[i, pl.ds(c * _L, _L)]
            vtab[i, pl.ds(c * _L, _L)] = jnp.maximum(3.0 + wv, 0.0)
            vtab[_N + i, pl.ds(c * _L, _L)] = jnp.maximum(3.0 - wv, 0.0)
        return 0

    lax.fori_loop(0, _N, _build_v, 0, unroll=8)

    vmm = []
    for c in range(_CH):
        def _mm(i, carry, c=c):
            mn, mx = carry
            vv = vtab[i, pl.ds(c * _L, _L)]
            return jnp.minimum(mn, vv), jnp.maximum(mx, vv)

        vmm.append(lax.fori_loop(
            0, 2 * _N, _mm,
            (jnp.full((_L,), 1e9, jnp.float32), jnp.full((_L,), -1e9, jnp.float32))))

    def _ugroup(g, _):
        def _umax(i, accs):
            return tuple(
                jnp.maximum(accs[j], ut[pl.ds((i + (2 * _N // _ACC) * j) * _RPW + g * _L, _L)])
                for j in range(_ACC))

        hi_accs = lax.fori_loop(0, 2 * _N // _ACC, _umax, (zeros,) * _ACC, unroll=2)
        hi = hi_accs[0]
        for j in range(1, _ACC):
            hi = jnp.maximum(hi, hi_accs[j])
        lo = zeros

        def _ubis(_, carry):
            lo, hi = carry
            mid = 0.5 * (lo + hi)

            def _ucnt(i, accs):
                return tuple(
                    accs[j] + jnp.where(
                        ut[pl.ds((i + (2 * _N // _ACC) * j) * _RPW + g * _L, _L)] <= mid, 1.0, 0.0)
                    for j in range(_ACC))

            accs = lax.fori_loop(0, 2 * _N // _ACC, _ucnt, (zeros,) * _ACC, unroll=2)
            cnt = accs[0]
            for j in range(1, _ACC):
                cnt = cnt + accs[j]
            ge = cnt >= kf
            return jnp.where(ge, lo, mid), jnp.where(ge, mid, hi)

        lo, hi = lax.fori_loop(0, _U_ITERS_SC, _ubis, (lo, hi))
        u64lo[pl.ds(g * _L, _L)] = lo
        u64hi[pl.ds(g * _L, _L)] = hi
        return 0

    lax.fori_loop(0, _RG, _ugroup, 0)

    def _rowgrp(rg, _):
      def _row(rl, _, rg=rg):
        r = rg * _L + rl
        rsp = _splat_i32(r)
        u_lo = plsc.load_gather(u64lo, [rsp])
        u_hi = plsc.load_gather(u64hi, [rsp])

        def _bcast(i, _):
            ubc[i] = plsc.load_gather(ut, [_splat_i32(i * _RPW) + rsp])
            return 0

        lax.fori_loop(0, 2 * _N, _bcast, 0, unroll=8)

        for c in range(_CH):
            vmn, vmx = vmm[c]
            lo0 = u_lo + vmn
            hi0 = u_hi + vmx
            width = jnp.maximum(hi0 - lo0, 1e-6)
            scale = jnp.float32(_NB) / width

            def _variant(variant, s_prev, c=c, lo0=lo0, scale=scale,
                         width=width, rsp=rsp, rl=rl):
                off = variant * _N

                def _zero(j, _):
                    hist[pl.ds(j * _L, _L)] = zeros
                    return 0

                lax.fori_loop(0, _NB, _zero, 0, unroll=8)

                def _scat(i, _):
                    row = lax.bitwise_and(i + off, 2 * _N - 1)
                    z = ubc[i] + vtab[row, pl.ds(c * _L, _L)]
                    zbuf[i] = z
                    q = (z - lo0) * scale
                    qi = jnp.clip(q.astype(jnp.int32), 0, _NB - 1)
                    plsc.addupdate_scatter(
                        hist, [lax.shift_left(qi, 4) + lane], ones)
                    return 0

                lax.fori_loop(0, 2 * _N, _scat, 0, unroll=8)

                def _cscan(g, carry):
                    cum, gsel, cbefore = carry
                    gb = g * _GS * _L

                    def _gsum(k, accs):
                        a0, a1 = accs
                        return (a0 + hist[pl.ds(gb + 2 * k * _L, _L)],
                                a1 + hist[pl.ds(gb + (2 * k + 1) * _L, _L)])

                    s0, s1 = lax.fori_loop(0, _GS // 2, _gsum, (zeros, zeros),
                                           unroll=4)
                    newcum = cum + s0 + s1
                    first = (newcum >= kf) & (gsel >= _NG)
                    gsel = jnp.where(first, g, gsel)
                    cbefore = jnp.where(first, cum, cbefore)
                    return newcum, gsel, cbefore

                _, gsel, cbefore = lax.fori_loop(
                    0, _NG, _cscan, (zeros, _splat_i32(_NG), zeros))

                def _fscan(k, carry):
                    cumf, bsel = carry
                    idx = gsel * _GS + k
                    hv = plsc.load_gather(hist, [lax.shift_left(idx, 4) + lane])
                    newcum = cumf + hv
                    first = (newcum >= kf) & (bsel >= _NB)
                    bsel = jnp.where(first, idx, bsel)
                    return newcum, bsel

                _, bsel = lax.fori_loop(
                    0, _GS, _fscan, (cbefore, _splat_i32(_NB)))

                t = lo0 + (bsel + 1).astype(jnp.float32) * (width * (1.0 / _NB))

                def _smin(i, accs):
                    return tuple(
                        accs[j] + jnp.minimum(zbuf[i + (2 * _N // _ACC) * j], t)
                        for j in range(_ACC))

                s_accs = lax.fori_loop(
                    0, 2 * _N // _ACC, _smin, (zeros,) * _ACC, unroll=2)
                s_min = s_accs[0]
                for j in range(1, _ACC):
                    s_min = s_min + s_accs[j]
                spike = (s_min - jnp.float32(2 * _N - _K) * t) * (1.0 / _K)

                @pl.when(variant == 1)
                def _():
                    out_v[rl, pl.ds(c * _L, _L)] = s_prev - spike

                return spike

            lax.fori_loop(0, 2, _variant, zeros)
        return 0

      lax.fori_loop(0, _L, _row, 0)
      pltpu.sync_copy(out_v, out_hbm.at[pl.ds(base + rg * _L, _L)])
      return 0

    lax.fori_loop(0, _RG, _rowgrp, 0)


@jax.jit
def kernel(inputp, W):
    out_sc = _sc_spike(inputp[_TC_B:].reshape(-1), W)
    out_tc = _tc_part(inputp[:_TC_B], W)
    return jnp.concatenate([out_tc, out_sc], axis=0)
